# baseline (speedup 1.0000x reference)
import functools

import jax
import jax.numpy as jnp
from jax import lax
from jax.experimental import pallas as pl
from jax.experimental.pallas import tpu as pltpu
from jax.experimental.pallas import tpu_sc as plsc

HIDDEN = 128
BATCH = 4
SEQ = 4096
EPS = 1e-3

NC = 2
NS = 16
NW = NC * NS
CHUNK = 128
N_CHUNKS = BATCH
L = 16
NH = HIDDEN // L
NG = CHUNK // L


def _rsqrt_newton(v):
    bits = plsc.bitcast(v, jnp.int32)
    y = plsc.bitcast(jnp.int32(0x5F3759DF) - (bits >> 1), jnp.float32)
    for _ in range(3):
        y = y * (1.5 - 0.5 * v * y * y)
    return y


def _tree8(vs):
    return ((vs[0] + vs[1]) + (vs[2] + vs[3])) + ((vs[4] + vs[5]) + (vs[6] + vs[7]))


def _sc_embed(token_idx, seg_f, word_emb, pos_slice, params):
    mesh = plsc.VectorSubcoreMesh(core_axis_name="c", subcore_axis_name="s")

    @functools.partial(
        pl.kernel,
        mesh=mesh,
        compiler_params=pltpu.CompilerParams(
            needs_layout_passes=False,
            skip_device_barrier=True,
            disable_bounds_checks=True,
            disable_semaphore_checks=True,
        ),
        out_type=jax.ShapeDtypeStruct((BATCH, SEQ, HIDDEN), jnp.float32),
        scratch_types=[
            pltpu.VMEM((N_CHUNKS, CHUNK), jnp.int32),
            pltpu.VMEM((N_CHUNKS, CHUNK), jnp.float32),
            pltpu.VMEM((N_CHUNKS, CHUNK, HIDDEN), jnp.float32),
            pltpu.VMEM((CHUNK, HIDDEN), jnp.float32),
            pltpu.VMEM((CHUNK, HIDDEN), jnp.float32),
            pltpu.VMEM((4, HIDDEN), jnp.float32),
            pltpu.VMEM((L, CHUNK), jnp.float32),
            pltpu.VMEM((L, CHUNK), jnp.float32),
            pltpu.VMEM((2, CHUNK), jnp.float32),
            pltpu.SemaphoreType.DMA,
            pltpu.SemaphoreType.DMA,
        ],
    )
    def k(table_hbm, idx_hbm, segf_hbm, pos_hbm, par_hbm, out_hbm,
          idx_v, segf_v, rows_v, xbuf, pos_v, params_v, psumT, qsumT, meaniv,
          gsem, wsem):
        wid = lax.axis_index("s") * NC + lax.axis_index("c")
        s0 = wid * CHUNK
        pltpu.sync_copy(idx_hbm.at[wid], idx_v)
        gathers = [
            pltpu.async_copy(table_hbm.at[idx_v.at[b]], rows_v.at[b], gsem)
            for b in range(N_CHUNKS)
        ]
        pltpu.sync_copy(segf_hbm.at[wid], segf_v)
        pltpu.sync_copy(par_hbm, params_v)
        pltpu.sync_copy(pos_hbm.at[pl.ds(s0, CHUNK)], pos_v)

        seg0 = [params_v[0, pl.ds(h * L, L)] for h in range(NH)]
        dseg = [params_v[1, pl.ds(h * L, L)] for h in range(NH)]
        gam = [params_v[2, pl.ds(h * L, L)] for h in range(NH)]
        bet = [params_v[3, pl.ds(h * L, L)] for h in range(NH)]
        inv_h = jnp.float32(1.0 / HIDDEN)
        lane = lax.iota(jnp.int32, L)
        zero16 = jnp.zeros((L,), jnp.int32)
        one16 = jnp.full((L,), 1, jnp.int32)

        @plsc.parallel_loop(0, CHUNK, step=1, unroll=4)
        def _fold_seg0(r):
            for h in range(NH):
                sl = pl.ds(h * L, L)
                pos_v[r, sl] = pos_v[r, sl] + seg0[h]

        writebacks = []
        for b in range(N_CHUNKS):
            gathers[b].wait()
            chunk = rows_v.at[b]
            bfull = jnp.full((L,), b, jnp.int32)

            @plsc.parallel_loop(0, CHUNK, step=1, unroll=8)
            def _add_row(r):
                rfull = jnp.full((L,), r, jnp.int32)
                sf = plsc.load_gather(segf_v, [bfull, rfull])
                xs = []
                for h in range(NH):
                    sl = pl.ds(h * L, L)
                    x = chunk[r, sl] + (pos_v[r, sl] + sf * dseg[h])
                    chunk[r, sl] = x
                    xs.append(x)
                s = _tree8(xs)
                q = _tree8([x * x for x in xs])
                plsc.store_scatter(psumT, [lane, rfull], s)
                plsc.store_scatter(qsumT, [lane, rfull], q)

            @plsc.parallel_loop(0, NG, step=1, unroll=2)
            def _stats(g):
                sl = pl.ds(g * L, L)
                svs = [psumT[kk, sl] for kk in range(L)]
                qvs = [qsumT[kk, sl] for kk in range(L)]
                tot = _tree8([svs[2 * i] + svs[2 * i + 1] for i in range(8)])
                totq = _tree8([qvs[2 * i] + qvs[2 * i + 1] for i in range(8)])
                mean = tot * inv_h
                var = totq * inv_h - mean * mean
                meaniv[0, sl] = mean
                meaniv[1, sl] = _rsqrt_newton(var + EPS)

            @plsc.parallel_loop(0, CHUNK, step=1, unroll=8)
            def _norm_row(r):
                rfull = jnp.full((L,), r, jnp.int32)
                m = plsc.load_gather(meaniv, [zero16, rfull])
                iv = plsc.load_gather(meaniv, [one16, rfull])
                for h in range(NH):
                    sl = pl.ds(h * L, L)
                    chunk[r, sl] = (chunk[r, sl] - m) * iv * gam[h] + bet[h]

            writebacks.append(
                pltpu.async_copy(chunk, out_hbm.at[b, pl.ds(s0, CHUNK)], wsem)
            )
        for wb in writebacks:
            wb.wait()

    return k(word_emb, token_idx, seg_f, pos_slice, params)


def kernel(token, segment, word_emb, seg_emb, pos_emb, gamma, beta):
    tok = token.astype(jnp.int32).reshape(BATCH, NW, CHUNK).swapaxes(0, 1)
    seg_f = segment.astype(jnp.float32).reshape(BATCH, NW, CHUNK).swapaxes(0, 1)
    params = jnp.stack([seg_emb[0], seg_emb[1] - seg_emb[0], gamma, beta])
    return _sc_embed(tok, seg_f, word_emb, pos_emb[:SEQ], params)


# speedup vs baseline: 1.1349x; 1.1349x over previous
import functools

import jax
import jax.numpy as jnp
from jax import lax
from jax.experimental import pallas as pl
from jax.experimental.pallas import tpu as pltpu
from jax.experimental.pallas import tpu_sc as plsc

HIDDEN = 128
BATCH = 4
SEQ = 4096
EPS = 1e-3

NC = 2
NS = 16
NW = NC * NS
CHUNK = 128
N_CHUNKS = BATCH
L = 16
NH = HIDDEN // L
NG = CHUNK // L


def _rsqrt_newton(v):
    bits = plsc.bitcast(v, jnp.int32)
    y = plsc.bitcast(jnp.int32(0x5F3759DF) - (bits >> 1), jnp.float32)
    for _ in range(3):
        y = y * (1.5 - 0.5 * v * y * y)
    return y


def _tree8(vs):
    return ((vs[0] + vs[1]) + (vs[2] + vs[3])) + ((vs[4] + vs[5]) + (vs[6] + vs[7]))


def _sc_embed(token_idx, seg_f, word_emb, pos_slice, params):
    mesh = plsc.VectorSubcoreMesh(core_axis_name="c", subcore_axis_name="s")

    @functools.partial(
        pl.kernel,
        mesh=mesh,
        compiler_params=pltpu.CompilerParams(
            needs_layout_passes=False,
            skip_device_barrier=True,
            disable_bounds_checks=True,
            disable_semaphore_checks=True,
        ),
        out_type=jax.ShapeDtypeStruct((BATCH, SEQ, HIDDEN), jnp.float32),
        scratch_types=[
            pltpu.VMEM((N_CHUNKS, CHUNK), jnp.int32),
            pltpu.VMEM((N_CHUNKS, CHUNK), jnp.float32),
            pltpu.VMEM((N_CHUNKS, CHUNK, HIDDEN), jnp.float32),
            pltpu.VMEM((CHUNK, HIDDEN), jnp.float32),
            pltpu.VMEM((4, HIDDEN), jnp.float32),
            pltpu.VMEM((L, CHUNK), jnp.float32),
            pltpu.VMEM((L, CHUNK), jnp.float32),
            pltpu.VMEM((2, CHUNK), jnp.float32),
            pltpu.SemaphoreType.DMA,
            pltpu.SemaphoreType.DMA,
        ],
    )
    def k(table_hbm, idx_hbm, segf_hbm, pos_hbm, par_hbm, out_hbm,
          idx_v, segf_v, rows_v, pos_v, params_v, psumT, qsumT, meaniv,
          gsem, wsem):
        wid = lax.axis_index("s") * NC + lax.axis_index("c")
        s0 = wid * CHUNK
        pltpu.sync_copy(idx_hbm.at[wid], idx_v)
        gathers = [
            pltpu.async_copy(table_hbm.at[idx_v.at[b]], rows_v.at[b], gsem)
            for b in range(N_CHUNKS)
        ]
        pltpu.sync_copy(segf_hbm.at[wid], segf_v)
        pltpu.sync_copy(par_hbm, params_v)
        pltpu.sync_copy(pos_hbm.at[pl.ds(s0, CHUNK)], pos_v)

        seg0 = [params_v[0, pl.ds(h * L, L)] for h in range(NH)]
        dseg = [params_v[1, pl.ds(h * L, L)] for h in range(NH)]
        gam = [params_v[2, pl.ds(h * L, L)] for h in range(NH)]
        bet = [params_v[3, pl.ds(h * L, L)] for h in range(NH)]
        inv_h = jnp.float32(1.0 / HIDDEN)
        lane = lax.iota(jnp.int32, L)
        zero16 = jnp.zeros((L,), jnp.int32)
        one16 = jnp.full((L,), 1, jnp.int32)

        @plsc.parallel_loop(0, CHUNK, step=1, unroll=4)
        def _fold_seg0(r):
            for h in range(NH):
                sl = pl.ds(h * L, L)
                pos_v[r, sl] = pos_v[r, sl] + seg0[h]

        writebacks = []
        for b in range(N_CHUNKS):
            gathers[b].wait()
            chunk = rows_v.at[b]
            bfull = jnp.full((L,), b, jnp.int32)

            @plsc.parallel_loop(0, CHUNK, step=1, unroll=4)
            def _add_row(r):
                rfull = jnp.full((L,), r, jnp.int32)
                sf = plsc.load_gather(segf_v, [bfull, rfull])
                xs = []
                for h in range(NH):
                    sl = pl.ds(h * L, L)
                    x = chunk[r, sl] + (pos_v[r, sl] + sf * dseg[h])
                    chunk[r, sl] = x
                    xs.append(x)
                s = _tree8(xs)
                q = _tree8([x * x for x in xs])
                plsc.store_scatter(psumT, [lane, rfull], s)
                plsc.store_scatter(qsumT, [lane, rfull], q)

            @plsc.parallel_loop(0, NG, step=1, unroll=2)
            def _stats(g):
                sl = pl.ds(g * L, L)
                svs = [psumT[kk, sl] for kk in range(L)]
                qvs = [qsumT[kk, sl] for kk in range(L)]
                tot = _tree8([svs[2 * i] + svs[2 * i + 1] for i in range(8)])
                totq = _tree8([qvs[2 * i] + qvs[2 * i + 1] for i in range(8)])
                mean = tot * inv_h
                var = totq * inv_h - mean * mean
                meaniv[0, sl] = mean
                meaniv[1, sl] = _rsqrt_newton(var + EPS)

            @plsc.parallel_loop(0, CHUNK, step=1, unroll=4)
            def _norm_row(r):
                rfull = jnp.full((L,), r, jnp.int32)
                m = plsc.load_gather(meaniv, [zero16, rfull])
                iv = plsc.load_gather(meaniv, [one16, rfull])
                for h in range(NH):
                    sl = pl.ds(h * L, L)
                    chunk[r, sl] = (chunk[r, sl] - m) * iv * gam[h] + bet[h]

            writebacks.append(
                pltpu.async_copy(chunk, out_hbm.at[b, pl.ds(s0, CHUNK)], wsem)
            )
        for wb in writebacks:
            wb.wait()

    return k(word_emb, token_idx, seg_f, pos_slice, params)


def kernel(token, segment, word_emb, seg_emb, pos_emb, gamma, beta):
    tok = token.astype(jnp.int32).reshape(BATCH, NW, CHUNK).swapaxes(0, 1)
    seg_f = segment.astype(jnp.float32).reshape(BATCH, NW, CHUNK).swapaxes(0, 1)
    params = jnp.stack([seg_emb[0], seg_emb[1] - seg_emb[0], gamma, beta])
    return _sc_embed(tok, seg_f, word_emb, pos_emb[:SEQ], params)
